# baseline (speedup 1.0000x reference)
import functools

import jax
import jax.numpy as jnp
from jax import lax
from jax.experimental import pallas as pl
from jax.experimental.pallas import tpu as pltpu
from jax.experimental.pallas import tpu_sc as plsc

BATCH = 16384
N_PERM = 200
N_EMBED = 64
TCOLS = 256
PGROUP = 8


def _sc_embed(x_t, cpt, post):
    info = plsc.get_sparse_core_info()
    nw = info.num_cores * info.num_subcores
    bw = BATCH // nw
    n_bv = bw // 16

    mesh = plsc.VectorSubcoreMesh(core_axis_name="c", subcore_axis_name="s")

    @functools.partial(
        pl.kernel,
        out_type=jax.ShapeDtypeStruct((N_PERM, N_EMBED, BATCH), jnp.float32),
        mesh=mesh,
        scratch_types=[
            pltpu.VMEM((N_EMBED * TCOLS,), jnp.float32),
            pltpu.VMEM((N_EMBED, TCOLS), jnp.float32),
            pltpu.VMEM((PGROUP, bw), jnp.int32),
            pltpu.VMEM((N_EMBED, bw), jnp.float32),
            pltpu.VMEM((N_EMBED, bw), jnp.float32),
            pltpu.SemaphoreType.DMA,
            pltpu.SemaphoreType.DMA,
        ],
        compiler_params=pltpu.CompilerParams(needs_layout_passes=False),
    )
    def k(x_hbm, cpt_hbm, post_hbm, out_hbm, cpt_v, post_v, x_v,
          stage0, stage1, sem_o0, sem_o1):
        wid = lax.axis_index("s") * info.num_cores + lax.axis_index("c")
        b0 = wid * bw
        stage = (stage0, stage1)
        sem_o = (sem_o0, sem_o1)

        pltpu.sync_copy(cpt_hbm, cpt_v)
        pltpu.sync_copy(post_hbm, post_v)

        def compute(p, pp, sb):
            xvs = [x_v[pp, pl.ds(bv * 16, 16)] for bv in range(n_bv)]

            @plsc.parallel_loop(0, N_EMBED, unroll=2)
            def e_body(e):
                pose = post_v[e, pl.ds(p, 16)][0]
                ebase = jnp.broadcast_to(e * TCOLS, (16,)).astype(jnp.int32)
                for bv in range(n_bv):
                    g = plsc.load_gather(cpt_v, [ebase + xvs[bv]])
                    stage[sb][e, pl.ds(bv * 16, 16)] = g + pose

        def out_copy(p, sb):
            return pltpu.make_async_copy(
                stage[sb],
                out_hbm.at[p, :, pl.ds(b0, bw)],
                sem_o[sb],
            )

        def group(g, carry):
            p0 = g * PGROUP
            pltpu.sync_copy(x_hbm.at[pl.ds(p0, PGROUP), pl.ds(b0, bw)], x_v)
            for pp in range(PGROUP):
                p = p0 + pp
                sb = pp % 2

                @pl.when(p >= 2)
                def _():
                    out_copy(p, sb).wait()
                compute(p, pp, sb)
                out_copy(p, sb).start()
            return carry

        lax.fori_loop(0, N_PERM // PGROUP, group, 0)
        out_copy(N_PERM - 2, 0).wait()
        out_copy(N_PERM - 1, 1).wait()

    return k(x_t, cpt, post)


def kernel(x, c_perm, pos_embedding):
    x_t = x.T.astype(jnp.int32)
    pad = ((0, 0), (0, TCOLS - N_PERM))
    cpt = jnp.pad(c_perm.T, pad).reshape(-1)
    post = jnp.pad(pos_embedding.T, pad)
    out = _sc_embed(x_t, cpt, post)
    return out.transpose(2, 0, 1)


# speedup vs baseline: 2.4388x; 2.4388x over previous
import functools

import jax
import jax.numpy as jnp
from jax import lax
from jax.experimental import pallas as pl
from jax.experimental.pallas import tpu as pltpu
from jax.experimental.pallas import tpu_sc as plsc

BATCH = 16384
N_PERM = 200
N_EMBED = 64
TCOLS = 256
PGROUP = 8


def _sc_embed(x_t, cpt, post):
    info = plsc.get_sparse_core_info()
    nw = info.num_cores * info.num_subcores
    bw = BATCH // nw
    n_bv = bw // 16

    mesh = plsc.VectorSubcoreMesh(core_axis_name="c", subcore_axis_name="s")

    @functools.partial(
        pl.kernel,
        out_type=jax.ShapeDtypeStruct((N_PERM, N_EMBED, BATCH), jnp.float32),
        mesh=mesh,
        scratch_types=[
            pltpu.VMEM((N_EMBED * TCOLS,), jnp.float32),
            pltpu.VMEM((N_EMBED, TCOLS), jnp.float32),
            pltpu.VMEM((PGROUP, bw), jnp.int32),
            pltpu.VMEM((N_EMBED, bw), jnp.float32),
            pltpu.VMEM((N_EMBED, bw), jnp.float32),
            pltpu.SemaphoreType.DMA,
            pltpu.SemaphoreType.DMA,
        ],
        compiler_params=pltpu.CompilerParams(needs_layout_passes=False),
    )
    def k(x_hbm, cpt_hbm, post_hbm, out_hbm, cpt_v, post_v, x_v,
          stage0, stage1, sem_o0, sem_o1):
        wid = lax.axis_index("s") * info.num_cores + lax.axis_index("c")
        b0 = wid * bw
        stage = (stage0, stage1)
        sem_o = (sem_o0, sem_o1)

        pltpu.sync_copy(cpt_hbm, cpt_v)
        pltpu.sync_copy(post_hbm, post_v)

        def compute(p, pp, sb):
            xvs = [x_v[pp, pl.ds(bv * 16, 16)] for bv in range(n_bv)]

            @plsc.parallel_loop(0, N_EMBED, unroll=1)
            def e_body(e):
                pose = post_v[e, pl.ds(p, 16)][0]
                ebase = jnp.broadcast_to(e * TCOLS, (16,)).astype(jnp.int32)
                for bv in range(n_bv):
                    g = plsc.load_gather(cpt_v, [ebase + xvs[bv]])
                    stage[sb][e, pl.ds(bv * 16, 16)] = g + pose

        def out_copy(p, sb):
            return pltpu.make_async_copy(
                stage[sb],
                out_hbm.at[p, :, pl.ds(b0, bw)],
                sem_o[sb],
            )

        def group(g, carry):
            p0 = g * PGROUP
            pltpu.sync_copy(x_hbm.at[pl.ds(p0, PGROUP), pl.ds(b0, bw)], x_v)
            for pp in range(PGROUP):
                p = p0 + pp
                sb = pp % 2

                @pl.when(p >= 2)
                def _():
                    out_copy(p, sb).wait()
                compute(p, pp, sb)
                out_copy(p, sb).start()
            return carry

        lax.fori_loop(0, N_PERM // PGROUP, group, 0)
        out_copy(N_PERM - 2, 0).wait()
        out_copy(N_PERM - 1, 1).wait()

    return k(x_t, cpt, post)


def kernel(x, c_perm, pos_embedding):
    x_t = x.T.astype(jnp.int32)
    pad = ((0, 0), (0, TCOLS - N_PERM))
    cpt = jnp.pad(c_perm.T, pad).reshape(-1)
    post = jnp.pad(pos_embedding.T, pad)
    out = _sc_embed(x_t, cpt, post)
    return out.transpose(2, 0, 1)
